# baseline (speedup 1.0000x reference)
import functools

import jax
import jax.numpy as jnp
from jax import lax
from jax.experimental import pallas as pl
from jax.experimental.pallas import tpu as pltpu
from jax.experimental.pallas import tpu_sc as plsc

_T, _D, _E, _K, _FF, _G, _TG = 2048, 1024, 16, 8, 512, 4, 2
_EPG = _E // _G
_NC, _NS = 2, 16
_NW = _NC * _NS
_NACT = 16
_TPW = _T // _NACT
_TILES = _TPW // 16


def _gate_body(x_ref, gw_ref, bias_ref, scores_ref, sfc_ref, xb_ref):
    x = x_ref[...]
    logits = lax.dot_general(
        x, gw_ref[...], (((1,), (1,)), ((), ())),
        preferred_element_type=jnp.float32)
    scores = 1.0 / (1.0 + jnp.exp(-logits))
    scores_ref[...] = scores.T
    sfc_ref[...] = scores.T + bias_ref[...]
    xb_ref[...] = x.astype(jnp.bfloat16)


def _route_sc_body(scores_hbm, sfc_hbm, dw_hbm, sc_s, sfc_s, out_s):
    wid = lax.axis_index("s") * _NC + lax.axis_index("c")

    @pl.when(wid < _NACT)
    def _():
        _route_worker(wid, scores_hbm, sfc_hbm, dw_hbm, sc_s, sfc_s, out_s)


def _route_worker(wid, scores_hbm, sfc_hbm, dw_hbm, sc_s, sfc_s, out_s):
    base = wid * _TPW
    pltpu.sync_copy(scores_hbm.at[:, pl.ds(base, _TPW)], sc_s)
    pltpu.sync_copy(sfc_hbm.at[:, pl.ds(base, _TPW)], sfc_s)

    for t in range(_TILES):
        sl = pl.ds(t * 16, 16)
        c = [sfc_s[j, sl] for j in range(_E)]
        gsums = []
        for g in range(_G):
            c4 = c[4 * g:4 * g + 4]
            hi01, lo01 = jnp.maximum(c4[0], c4[1]), jnp.minimum(c4[0], c4[1])
            hi23, lo23 = jnp.maximum(c4[2], c4[3]), jnp.minimum(c4[2], c4[3])
            top1 = jnp.maximum(hi01, hi23)
            second = jnp.maximum(jnp.minimum(hi01, hi23),
                                 jnp.where(hi01 >= hi23, lo01, lo23))
            gsums.append(top1 + second)
        msel = []
        for g in range(_G):
            beats = jnp.zeros((16,), jnp.float32)
            for j in range(_G):
                if j == g:
                    continue
                b = (gsums[j] >= gsums[g]) if j < g else (gsums[j] > gsums[g])
                beats = beats + jnp.where(b, 1.0, 0.0)
            msel.append(jnp.where(beats < float(_TG), 1.0, 0.0))
        w = [msel[j // _EPG] * sc_s[j, sl] for j in range(_E)]
        denom = w[0]
        for j in range(1, _E):
            denom = denom + w[j]
        denom = denom + 1e-20
        for j in range(_E):
            out_s[j, sl] = w[j] / denom

    pltpu.sync_copy(out_s, dw_hbm.at[:, pl.ds(base, _TPW)])


def _expert_body(x_ref, wg_ref, wu_ref, wd_ref, dw_ref, out_ref):
    e = pl.program_id(0)
    wg = wg_ref[0].astype(jnp.bfloat16)
    wu = wu_ref[0].astype(jnp.bfloat16)
    wd = wd_ref[0].astype(jnp.bfloat16)
    dwc = dw_ref[0].T
    for i in range(4):
        sl = pl.ds(i * (_T // 4), _T // 4)
        xh = x_ref[sl, :]
        g = lax.dot_general(xh, wg, (((1,), (1,)), ((), ())),
                            preferred_element_type=jnp.float32)
        u = lax.dot_general(xh, wu, (((1,), (1,)), ((), ())),
                            preferred_element_type=jnp.float32)
        h = (g / (1.0 + jnp.exp(-g))) * (u * dwc[i * (_T // 4):
                                                 (i + 1) * (_T // 4), :])
        o = lax.dot_general(h.astype(jnp.bfloat16), wd,
                            (((1,), (1,)), ((), ())),
                            preferred_element_type=jnp.float32)

        @pl.when(e == 0)
        def _():
            out_ref[sl, :] = o

        @pl.when(e != 0)
        def _():
            out_ref[sl, :] = out_ref[sl, :] + o


def kernel(hidden_states, gate_weight, e_score_correction_bias,
           w_gate, w_up, w_down):
    x32 = hidden_states.astype(jnp.float32)
    scores_t, sfc_t, xb = pl.pallas_call(
        _gate_body,
        out_shape=(
            jax.ShapeDtypeStruct((_E, _T), jnp.float32),
            jax.ShapeDtypeStruct((_E, _T), jnp.float32),
            jax.ShapeDtypeStruct((_T, _D), jnp.bfloat16),
        ),
    )(x32, gate_weight, e_score_correction_bias.reshape(_E, 1))

    route = functools.partial(
        pl.kernel,
        mesh=plsc.VectorSubcoreMesh(core_axis_name="c", subcore_axis_name="s"),
        out_type=jax.ShapeDtypeStruct((_E, _T), jnp.float32),
        scratch_types=[
            pltpu.VMEM((_E, _TPW), jnp.float32),
            pltpu.VMEM((_E, _TPW), jnp.float32),
            pltpu.VMEM((_E, _TPW), jnp.float32),
        ],
    )(_route_sc_body)
    dw_t = route(scores_t, sfc_t).reshape(_E, 1, _T)
    out = pl.pallas_call(
        _expert_body,
        grid=(_E,),
        in_specs=[
            pl.BlockSpec((_T, _D), lambda e: (0, 0)),
            pl.BlockSpec((1, _FF, _D), lambda e: (e, 0, 0)),
            pl.BlockSpec((1, _FF, _D), lambda e: (e, 0, 0)),
            pl.BlockSpec((1, _D, _FF), lambda e: (e, 0, 0)),
            pl.BlockSpec((1, 1, _T), lambda e: (e, 0, 0)),
        ],
        out_specs=pl.BlockSpec((_T, _D), lambda e: (0, 0)),
        out_shape=jax.ShapeDtypeStruct((_T, _D), jnp.float32),
        compiler_params=pltpu.CompilerParams(
            dimension_semantics=("arbitrary",)),
    )(xb, w_gate, w_up, w_down, dw_t)
    return out


# speedup vs baseline: 1.0394x; 1.0394x over previous
import functools

import jax
import jax.numpy as jnp
from jax import lax
from jax.experimental import pallas as pl
from jax.experimental.pallas import tpu as pltpu
from jax.experimental.pallas import tpu_sc as plsc

_T, _D, _E, _K, _FF, _G, _TG = 2048, 1024, 16, 8, 512, 4, 2
_EPG = _E // _G
_NC, _NS = 2, 16
_NW = _NC * _NS
_NACT = 16
_TPW = _T // _NACT
_TILES = _TPW // 16


def _gate_body(x_ref, gw_ref, bias_ref, scores_ref, sfc_ref, xb_ref):
    x = x_ref[...]
    logits = lax.dot_general(
        x, gw_ref[...], (((1,), (1,)), ((), ())),
        preferred_element_type=jnp.float32)
    scores = 1.0 / (1.0 + jnp.exp(-logits))
    scores_ref[...] = scores.T
    sfc_ref[...] = scores.T + bias_ref[...]
    xb_ref[...] = x.astype(jnp.bfloat16)


def _route_sc_body(scores_hbm, sfc_hbm, dw_hbm, sc_s, sfc_s, out_s):
    wid = lax.axis_index("s") * _NC + lax.axis_index("c")

    @pl.when(wid < _NACT)
    def _():
        _route_worker(wid, scores_hbm, sfc_hbm, dw_hbm, sc_s, sfc_s, out_s)


def _route_worker(wid, scores_hbm, sfc_hbm, dw_hbm, sc_s, sfc_s, out_s):
    base = wid * _TPW
    pltpu.sync_copy(scores_hbm.at[:, pl.ds(base, _TPW)], sc_s)
    pltpu.sync_copy(sfc_hbm.at[:, pl.ds(base, _TPW)], sfc_s)

    for t in range(_TILES):
        sl = pl.ds(t * 16, 16)
        c = [sfc_s[j, sl] for j in range(_E)]
        gsums = []
        for g in range(_G):
            c4 = c[4 * g:4 * g + 4]
            hi01, lo01 = jnp.maximum(c4[0], c4[1]), jnp.minimum(c4[0], c4[1])
            hi23, lo23 = jnp.maximum(c4[2], c4[3]), jnp.minimum(c4[2], c4[3])
            top1 = jnp.maximum(hi01, hi23)
            second = jnp.maximum(jnp.minimum(hi01, hi23),
                                 jnp.where(hi01 >= hi23, lo01, lo23))
            gsums.append(top1 + second)
        msel = []
        for g in range(_G):
            beats = jnp.zeros((16,), jnp.float32)
            for j in range(_G):
                if j == g:
                    continue
                b = (gsums[j] >= gsums[g]) if j < g else (gsums[j] > gsums[g])
                beats = beats + jnp.where(b, 1.0, 0.0)
            msel.append(jnp.where(beats < float(_TG), 1.0, 0.0))
        w = [msel[j // _EPG] * sc_s[j, sl] for j in range(_E)]
        denom = w[0]
        for j in range(1, _E):
            denom = denom + w[j]
        denom = denom + 1e-20
        for j in range(_E):
            out_s[j, sl] = w[j] / denom

    pltpu.sync_copy(out_s, dw_hbm.at[:, pl.ds(base, _TPW)])


def _expert_body(x_ref, wg_ref, wu_ref, wd_ref, dw_ref, out_ref):
    e = pl.program_id(0)
    wg = wg_ref[0].astype(jnp.bfloat16)
    wu = wu_ref[0].astype(jnp.bfloat16)
    wd = wd_ref[0].astype(jnp.bfloat16)
    dwc = dw_ref[0].T
    for i in range(2):
        sl = pl.ds(i * (_T // 2), _T // 2)
        xh = x_ref[sl, :]
        g = lax.dot_general(xh, wg, (((1,), (1,)), ((), ())),
                            preferred_element_type=jnp.float32)
        u = lax.dot_general(xh, wu, (((1,), (1,)), ((), ())),
                            preferred_element_type=jnp.float32)
        h = (g / (1.0 + jnp.exp(-g))) * (u * dwc[i * (_T // 2):
                                                 (i + 1) * (_T // 2), :])
        o = lax.dot_general(h.astype(jnp.bfloat16), wd,
                            (((1,), (1,)), ((), ())),
                            preferred_element_type=jnp.float32)

        @pl.when(e == 0)
        def _():
            out_ref[sl, :] = o

        @pl.when(e != 0)
        def _():
            out_ref[sl, :] = out_ref[sl, :] + o


def kernel(hidden_states, gate_weight, e_score_correction_bias,
           w_gate, w_up, w_down):
    x32 = hidden_states.astype(jnp.float32)
    scores_t, sfc_t, xb = pl.pallas_call(
        _gate_body,
        out_shape=(
            jax.ShapeDtypeStruct((_E, _T), jnp.float32),
            jax.ShapeDtypeStruct((_E, _T), jnp.float32),
            jax.ShapeDtypeStruct((_T, _D), jnp.bfloat16),
        ),
    )(x32, gate_weight, e_score_correction_bias.reshape(_E, 1))

    route = functools.partial(
        pl.kernel,
        mesh=plsc.VectorSubcoreMesh(core_axis_name="c", subcore_axis_name="s"),
        out_type=jax.ShapeDtypeStruct((_E, _T), jnp.float32),
        scratch_types=[
            pltpu.VMEM((_E, _TPW), jnp.float32),
            pltpu.VMEM((_E, _TPW), jnp.float32),
            pltpu.VMEM((_E, _TPW), jnp.float32),
        ],
    )(_route_sc_body)
    dw_t = route(scores_t, sfc_t).reshape(_E, 1, _T)
    out = pl.pallas_call(
        _expert_body,
        grid=(_E,),
        in_specs=[
            pl.BlockSpec((_T, _D), lambda e: (0, 0)),
            pl.BlockSpec((1, _FF, _D), lambda e: (e, 0, 0)),
            pl.BlockSpec((1, _FF, _D), lambda e: (e, 0, 0)),
            pl.BlockSpec((1, _D, _FF), lambda e: (e, 0, 0)),
            pl.BlockSpec((1, 1, _T), lambda e: (e, 0, 0)),
        ],
        out_specs=pl.BlockSpec((_T, _D), lambda e: (0, 0)),
        out_shape=jax.ShapeDtypeStruct((_T, _D), jnp.float32),
        compiler_params=pltpu.CompilerParams(
            dimension_semantics=("arbitrary",)),
    )(xb, w_gate, w_up, w_down, dw_t)
    return out
